# baseline (speedup 1.0000x reference)
import jax
import jax.numpy as jnp
from jax import lax
from jax.experimental import pallas as pl
from jax.experimental.pallas import tpu as pltpu
from jax.experimental.pallas import tpu_sc as plsc

B, C, H, W = 128, 768, 14, 14
HW = H * W
E, TOPK = 16, 2

KB = 28
NSTEP = HW // KB

NW = 32
RPW = B // NW


def _pool_body(x_ref, w1_ref, b1_ref, w2_ref, b2_ref, noise_ref,
               n_out, s_ref, m_ref):
    k = pl.program_id(0)
    xb = x_ref[...]
    ps = jnp.sum(xb, axis=0)
    pm = jnp.max(xb, axis=0)

    @pl.when(k == 0)
    def _init():
        s_ref[...] = ps
        m_ref[...] = pm

    @pl.when(k > 0)
    def _acc():
        s_ref[...] += ps
        m_ref[...] = jnp.maximum(m_ref[...], pm)

    @pl.when(k == NSTEP - 1)
    def _finish():
        f = s_ref[...] * (1.0 / HW) + m_ref[...]
        fb = f.astype(jnp.bfloat16)
        dn = (((1,), (1,)), ((), ()))
        z1 = lax.dot_general(
            fb, w1_ref[...].astype(jnp.bfloat16), dimension_numbers=dn,
            preferred_element_type=jnp.float32,
        ) + b1_ref[...]
        z2 = lax.dot_general(
            fb, w2_ref[...].astype(jnp.bfloat16), dimension_numbers=dn,
            preferred_element_type=jnp.float32,
        ) + b2_ref[...]
        nz = noise_ref[...].T
        n_out[...] = z1 + nz * jax.nn.softplus(z2)


def _argmax16(vals, iota):
    v, i = vals, iota
    for step in (8, 4, 2, 1):
        perm = jnp.bitwise_xor(iota, step)
        vp = v.at[perm].get(mode="promise_in_bounds")
        ip = i.at[perm].get(mode="promise_in_bounds")
        take = (vp > v) | ((vp == v) & (ip < i))
        v = jnp.where(take, vp, v)
        i = jnp.where(take, ip, i)
    return v, i


def _route_body(n_hbm, w_hbm, idx_hbm, rows_v, wv, idxv):
    wid = lax.axis_index("s") * 2 + lax.axis_index("c")
    base = wid * RPW
    pltpu.sync_copy(n_hbm.at[pl.ds(base, RPW)], rows_v)
    iota = jax.lax.iota(jnp.int32, E)
    idx_pack = jnp.zeros((E,), jnp.int32)
    for j in range(RPW):
        row = rows_v[j, :]
        v1, i1 = _argmax16(row, iota)
        masked = jnp.where(iota == i1, -jnp.inf, row)
        v2, i2 = _argmax16(masked, iota)
        e2 = jnp.exp(v2 - v1)
        denom = 1.0 + e2
        wrow = jnp.where(iota == i1, 1.0 / denom,
                         jnp.where(iota == i2, e2 / denom, 0.0))
        wv[j, :] = wrow
        idx_pack = jnp.where(iota == 2 * j, i1, idx_pack)
        idx_pack = jnp.where(iota == 2 * j + 1, i2, idx_pack)
    idxv[...] = idx_pack
    pltpu.sync_copy(wv, w_hbm.at[pl.ds(base, RPW)])
    pltpu.sync_copy(idxv.at[pl.ds(0, TOPK * RPW)],
                    idx_hbm.at[pl.ds(TOPK * base, TOPK * RPW)])


@jax.jit
def kernel(x, w1_w, w1_b, w2_w, w2_b, noise):
    xt = jnp.transpose(x, (2, 3, 0, 1)).reshape(HW, B, C)

    n = pl.pallas_call(
        _pool_body,
        grid=(NSTEP,),
        in_specs=[
            pl.BlockSpec((KB, B, C), lambda k: (k, 0, 0)),
            pl.BlockSpec((E, C), lambda k: (0, 0)),
            pl.BlockSpec((1, E), lambda k: (0, 0)),
            pl.BlockSpec((E, C), lambda k: (0, 0)),
            pl.BlockSpec((1, E), lambda k: (0, 0)),
            pl.BlockSpec((E, B), lambda k: (0, 0)),
        ],
        out_specs=pl.BlockSpec((B, E), lambda k: (0, 0)),
        out_shape=jax.ShapeDtypeStruct((B, E), jnp.float32),
        scratch_shapes=[
            pltpu.VMEM((B, C), jnp.float32),
            pltpu.VMEM((B, C), jnp.float32),
        ],
    )(xt, w1_w, w1_b.reshape(1, E), w2_w, w2_b.reshape(1, E), noise.T)

    mesh = plsc.VectorSubcoreMesh(core_axis_name="c", subcore_axis_name="s")
    w, idx_flat = pl.kernel(
        _route_body,
        mesh=mesh,
        out_type=[
            jax.ShapeDtypeStruct((B, E), jnp.float32),
            jax.ShapeDtypeStruct((B * TOPK,), jnp.int32),
        ],
        scratch_types=[
            pltpu.VMEM((RPW, E), jnp.float32),
            pltpu.VMEM((RPW, E), jnp.float32),
            pltpu.VMEM((E,), jnp.int32),
        ],
    )(n)
    return (w, idx_flat.reshape(B, TOPK))


# speedup vs baseline: 1.4889x; 1.4889x over previous
import jax
import jax.numpy as jnp
from jax import lax
from jax.experimental import pallas as pl
from jax.experimental.pallas import tpu as pltpu

B, C, H, W = 128, 768, 14, 14
HW = H * W
E, TOPK = 16, 2

KB = 14
NSTEP = HW // KB


def _gate_body(x_ref, w1_ref, b1_ref, w2_ref, b2_ref, noise_ref,
               w_out, idx_out, s_ref, m_ref):
    k = pl.program_id(0)
    xb = x_ref[...]
    ps = jnp.sum(xb, axis=0)
    pm = jnp.max(xb, axis=0)

    @pl.when(k == 0)
    def _init():
        s_ref[...] = ps
        m_ref[...] = pm

    @pl.when(k > 0)
    def _acc():
        s_ref[...] += ps
        m_ref[...] = jnp.maximum(m_ref[...], pm)

    @pl.when(k == NSTEP - 1)
    def _finish():
        f = s_ref[...] * (1.0 / HW) + m_ref[...]
        fb = f.astype(jnp.bfloat16)
        dn = (((1,), (1,)), ((), ()))
        z1 = lax.dot_general(
            fb, w1_ref[...].astype(jnp.bfloat16), dimension_numbers=dn,
            preferred_element_type=jnp.float32,
        ) + b1_ref[...]
        z2 = lax.dot_general(
            fb, w2_ref[...].astype(jnp.bfloat16), dimension_numbers=dn,
            preferred_element_type=jnp.float32,
        ) + b2_ref[...]

        n1 = z1
        n2 = z2
        nz = noise_ref[...].T
        n = n1 + nz * jax.nn.softplus(n2)

        iota = lax.broadcasted_iota(jnp.int32, (B, E), 1)
        v1 = jnp.max(n, axis=1, keepdims=True)
        i1 = jnp.min(jnp.where(n == v1, iota, E), axis=1, keepdims=True)
        masked = jnp.where(iota == i1, -jnp.inf, n)
        v2 = jnp.max(masked, axis=1, keepdims=True)
        i2 = jnp.min(jnp.where(masked == v2, iota, E), axis=1, keepdims=True)

        e2 = jnp.exp(v2 - v1)
        denom = 1.0 + e2
        w_out[...] = jnp.where(
            iota == i1, 1.0 / denom,
            jnp.where(iota == i2, e2 / denom, 0.0))
        idx_out[...] = jnp.concatenate([i1, i2], axis=1)


@jax.jit
def kernel(x, w1_w, w1_b, w2_w, w2_b, noise):
    xt = jnp.transpose(x, (2, 3, 0, 1)).reshape(HW, B, C)

    grid = (NSTEP,)
    w, idx = pl.pallas_call(
        _gate_body,
        grid=grid,
        in_specs=[
            pl.BlockSpec((KB, B, C), lambda k: (k, 0, 0)),
            pl.BlockSpec((E, C), lambda k: (0, 0)),
            pl.BlockSpec((1, E), lambda k: (0, 0)),
            pl.BlockSpec((E, C), lambda k: (0, 0)),
            pl.BlockSpec((1, E), lambda k: (0, 0)),
            pl.BlockSpec((E, B), lambda k: (0, 0)),
        ],
        out_specs=[
            pl.BlockSpec((B, E), lambda k: (0, 0)),
            pl.BlockSpec((B, TOPK), lambda k: (0, 0)),
        ],
        out_shape=[
            jax.ShapeDtypeStruct((B, E), jnp.float32),
            jax.ShapeDtypeStruct((B, TOPK), jnp.int32),
        ],
        scratch_shapes=[
            pltpu.VMEM((B, C), jnp.float32),
            pltpu.VMEM((B, C), jnp.float32),
        ],
    )(xt, w1_w, w1_b.reshape(1, E), w2_w, w2_b.reshape(1, E), noise.T)
    return (w, idx)


# speedup vs baseline: 1.6003x; 1.0748x over previous
import jax
import jax.numpy as jnp
from jax import lax
from jax.experimental import pallas as pl
from jax.experimental.pallas import tpu as pltpu

B, C, H, W = 128, 768, 14, 14
HW = H * W
E, TOPK = 16, 2

KB = 14
NSTEP = HW // KB // 2


def _gate_body(xa_ref, xb_ref, w1_ref, b1_ref, w2_ref, b2_ref, noise_ref,
               w_out, idx_out, s_ref, m_ref):
    k = pl.program_id(0)
    xa = xa_ref[...]
    xb = xb_ref[...]
    ps = jnp.sum(xa, axis=0) + jnp.sum(xb, axis=0)
    pm = jnp.maximum(jnp.max(xa, axis=0), jnp.max(xb, axis=0))

    @pl.when(k == 0)
    def _init():
        s_ref[...] = ps
        m_ref[...] = pm

    @pl.when(k > 0)
    def _acc():
        s_ref[...] += ps
        m_ref[...] = jnp.maximum(m_ref[...], pm)

    @pl.when(k == NSTEP - 1)
    def _finish():
        f = s_ref[...] * (1.0 / HW) + m_ref[...]
        fb = f.astype(jnp.bfloat16)
        dn = (((1,), (1,)), ((), ()))
        z1 = lax.dot_general(
            fb, w1_ref[...].astype(jnp.bfloat16), dimension_numbers=dn,
            preferred_element_type=jnp.float32,
        ) + b1_ref[...]
        z2 = lax.dot_general(
            fb, w2_ref[...].astype(jnp.bfloat16), dimension_numbers=dn,
            preferred_element_type=jnp.float32,
        ) + b2_ref[...]

        n1 = z1
        n2 = z2
        nz = noise_ref[...].T
        n = n1 + nz * jax.nn.softplus(n2)

        iota = lax.broadcasted_iota(jnp.int32, (B, E), 1)
        v1 = jnp.max(n, axis=1, keepdims=True)
        i1 = jnp.min(jnp.where(n == v1, iota, E), axis=1, keepdims=True)
        masked = jnp.where(iota == i1, -jnp.inf, n)
        v2 = jnp.max(masked, axis=1, keepdims=True)
        i2 = jnp.min(jnp.where(masked == v2, iota, E), axis=1, keepdims=True)

        e2 = jnp.exp(v2 - v1)
        denom = 1.0 + e2
        w_out[...] = jnp.where(
            iota == i1, 1.0 / denom,
            jnp.where(iota == i2, e2 / denom, 0.0))
        idx_out[...] = jnp.concatenate([i1, i2], axis=1)


@jax.jit
def kernel(x, w1_w, w1_b, w2_w, w2_b, noise):
    xt = jnp.transpose(x, (2, 3, 0, 1)).reshape(HW, B, C)

    grid = (NSTEP,)
    w, idx = pl.pallas_call(
        _gate_body,
        grid=grid,
        in_specs=[
            pl.BlockSpec((KB, B, C), lambda k: (k, 0, 0)),
            pl.BlockSpec((KB, B, C), lambda k: (k + NSTEP, 0, 0)),
            pl.BlockSpec((E, C), lambda k: (0, 0)),
            pl.BlockSpec((1, E), lambda k: (0, 0)),
            pl.BlockSpec((E, C), lambda k: (0, 0)),
            pl.BlockSpec((1, E), lambda k: (0, 0)),
            pl.BlockSpec((E, B), lambda k: (0, 0)),
        ],
        out_specs=[
            pl.BlockSpec((B, E), lambda k: (0, 0)),
            pl.BlockSpec((B, TOPK), lambda k: (0, 0)),
        ],
        out_shape=[
            jax.ShapeDtypeStruct((B, E), jnp.float32),
            jax.ShapeDtypeStruct((B, TOPK), jnp.int32),
        ],
        scratch_shapes=[
            pltpu.VMEM((B, C), jnp.float32),
            pltpu.VMEM((B, C), jnp.float32),
        ],
    )(xt, xt, w1_w, w1_b.reshape(1, E), w2_w, w2_b.reshape(1, E), noise.T)
    return (w, idx)
